# Initial kernel scaffold; baseline (speedup 1.0000x reference)
import jax
import jax.numpy as jnp
from jax.experimental import pallas as pl


def kernel(x, edge_index, W, att_src, att_dst, bias):
    raise NotImplementedError("write your pallas kernel here")


# speedup vs baseline: 14.1443x; 14.1443x over previous
import functools

import jax
import jax.numpy as jnp
from jax import lax
from jax.experimental import pallas as pl
from jax.experimental.pallas import tpu as pltpu
from jax.experimental.pallas import tpu_sc as plsc

_N = 10000
_E = 320000
_D = 128

_NC = 2
_NS = 16
_NW = _NC * _NS
_EPT = _E // _NW
_ECH = _EPT // 16
_NPT = _N // _NS
_NCH = _N // 16


def _bcast_lane(v, j):
    idx = jnp.full((16, 1), j, jnp.int32)
    dn = lax.GatherDimensionNumbers(offset_dims=(), collapsed_slice_dims=(0,),
                                    start_index_map=(0,))
    return lax.gather(v, idx, dn, slice_sizes=(1,),
                      mode=lax.GatherScatterMode.PROMISE_IN_BOUNDS)


def _dense_body(x_ref, w_ref, asv_ref, adv_ref, xw_ref, a_src_ref,
                a_dst_ref, m_ref):
    xw = jnp.dot(x_ref[...], w_ref[...], preferred_element_type=jnp.float32)
    xw_ref[...] = xw
    a_src = jnp.sum(xw * asv_ref[...][None, :], axis=1)
    a_dst = jnp.sum(xw * adv_ref[...][None, :], axis=1)
    a_src_ref[...] = a_src
    a_dst_ref[...] = a_dst
    m = jnp.max(a_src) + jnp.max(a_dst)
    m_ref[...] = jnp.full((16,), m, jnp.float32)


def _dense(x, w, att_src, att_dst):
    return pl.pallas_call(
        _dense_body,
        out_shape=(
            jax.ShapeDtypeStruct((_N, _D), jnp.float32),
            jax.ShapeDtypeStruct((_N,), jnp.float32),
            jax.ShapeDtypeStruct((_N,), jnp.float32),
            jax.ShapeDtypeStruct((16,), jnp.float32),
        ),
    )(x, w, att_src, att_dst)


def _edge_w_body(src_hbm, dst_hbm, asrc_hbm, adst_hbm, m_hbm,
                 w_hbm, dpart_hbm,
                 srcv, dstv, asv, adv, wv, denv, mv):
    cid = lax.axis_index("c")
    sid = lax.axis_index("s")
    gid = sid * _NC + cid
    base = gid * _EPT
    pltpu.sync_copy(src_hbm.at[pl.ds(base, _EPT)], srcv)
    pltpu.sync_copy(dst_hbm.at[pl.ds(base, _EPT)], dstv)
    pltpu.sync_copy(asrc_hbm, asv)
    pltpu.sync_copy(adst_hbm, adv)
    pltpu.sync_copy(m_hbm, mv)
    mvec = mv[...]

    zero = jnp.zeros((16,), jnp.float32)

    def zbody(i, c):
        denv[pl.ds(i * 16, 16)] = zero
        return c

    lax.fori_loop(0, _NCH, zbody, 0)

    def body(i, c):
        sidx = srcv[pl.ds(i * 16, 16)]
        didx = dstv[pl.ds(i * 16, 16)]
        s = plsc.load_gather(asv, [sidx])
        d = plsc.load_gather(adv, [didx])
        e = s + d
        e = jnp.maximum(e, e * 0.2)
        w = jnp.exp(e - mvec)
        wv[pl.ds(i * 16, 16)] = w
        plsc.addupdate_scatter(denv, [didx], w)
        return c

    lax.fori_loop(0, _ECH, body, 0)

    pltpu.sync_copy(wv, w_hbm.at[pl.ds(base, _EPT)])
    pltpu.sync_copy(denv, dpart_hbm.at[gid])


def _edge_w(src, dst, a_src, a_dst, m16):
    f = functools.partial(
        pl.kernel,
        mesh=plsc.VectorSubcoreMesh(core_axis_name="c", subcore_axis_name="s"),
        compiler_params=pltpu.CompilerParams(needs_layout_passes=False, use_tc_tiling_on_sc=False),
        out_type=(
            jax.ShapeDtypeStruct((_E,), jnp.float32),
            jax.ShapeDtypeStruct((_NW, _N), jnp.float32),
        ),
        scratch_types=[
            pltpu.VMEM((_EPT,), jnp.int32),
            pltpu.VMEM((_EPT,), jnp.int32),
            pltpu.VMEM((_N,), jnp.float32),
            pltpu.VMEM((_N,), jnp.float32),
            pltpu.VMEM((_EPT,), jnp.float32),
            pltpu.VMEM((_N,), jnp.float32),
            pltpu.VMEM((16,), jnp.float32),
        ],
    )(_edge_w_body)
    return f(src, dst, a_src, a_dst, m16)


def _alpha_body(dst_hbm, w_hbm, dpart_hbm, alpha_hbm,
                dstv, wv, denv, stagev, alphav):
    cid = lax.axis_index("c")
    sid = lax.axis_index("s")
    gid = sid * _NC + cid
    base = gid * _EPT
    pltpu.sync_copy(dst_hbm.at[pl.ds(base, _EPT)], dstv)
    pltpu.sync_copy(w_hbm.at[pl.ds(base, _EPT)], wv)

    pltpu.sync_copy(dpart_hbm.at[0], denv)

    def dacc(t, c):
        pltpu.sync_copy(dpart_hbm.at[t], stagev)

        def ib(i, cc):
            sl = pl.ds(i * 16, 16)
            denv[sl] = denv[sl] + stagev[sl]
            return cc

        lax.fori_loop(0, _NCH, ib, 0)
        return c

    lax.fori_loop(1, _NW, dacc, 0)

    def body(i, c):
        sl = pl.ds(i * 16, 16)
        didx = dstv[sl]
        deng = plsc.load_gather(denv, [didx])
        alphav[sl] = wv[sl] / deng
        return c

    lax.fori_loop(0, _ECH, body, 0)
    pltpu.sync_copy(alphav, alpha_hbm.at[pl.ds(base, _EPT)])


def _alpha(dst, w, dparts):
    f = functools.partial(
        pl.kernel,
        mesh=plsc.VectorSubcoreMesh(core_axis_name="c", subcore_axis_name="s"),
        compiler_params=pltpu.CompilerParams(needs_layout_passes=False, use_tc_tiling_on_sc=False),
        out_type=jax.ShapeDtypeStruct((_E,), jnp.float32),
        scratch_types=[
            pltpu.VMEM((_EPT,), jnp.int32),
            pltpu.VMEM((_EPT,), jnp.float32),
            pltpu.VMEM((_N,), jnp.float32),
            pltpu.VMEM((_N,), jnp.float32),
            pltpu.VMEM((_EPT,), jnp.float32),
        ],
    )(_alpha_body)
    return f(dst, w, dparts)


def _agg_body(src_hbm, dst_hbm, alpha_hbm, xw_hbm,
              pout_hbm,
              srcv, dstv, alphav, rows, acc_sp, sem):
    cid = lax.axis_index("c")
    sid = lax.axis_index("s")
    gid = sid * _NC + cid
    base = gid * _EPT
    pltpu.sync_copy(src_hbm.at[pl.ds(base, _EPT)], srcv)
    pltpu.sync_copy(dst_hbm.at[pl.ds(base, _EPT)], dstv)
    pltpu.sync_copy(alpha_hbm.at[pl.ds(base, _EPT)], alphav)

    zerov = jnp.zeros((16,), jnp.float32)
    for a in range(16):
        for b in range(_D // 16):
            rows[a, pl.ds(b * 16, 16)] = zerov
    nbase = sid * _NPT

    def zc(k, c):
        pltpu.sync_copy(rows, acc_sp.at[pl.ds(nbase + k * 16, 16), :])
        return c

    lax.fori_loop(0, _NPT // 16, zc, 0)
    pltpu.sync_copy(rows.at[pl.ds(0, _NPT % 16), :],
                    acc_sp.at[pl.ds(nbase + (_NPT // 16) * 16, _NPT % 16), :])
    plsc.subcore_barrier()

    def body(i, c):
        sidx = srcv[pl.ds(i * 16, 16)]
        didx = dstv[pl.ds(i * 16, 16)]
        alpha_v = alphav[pl.ds(i * 16, 16)]
        pltpu.async_copy(xw_hbm.at[sidx], rows, sem).wait()
        for j in range(16):
            aj = _bcast_lane(alpha_v, j)
            for r in range(_D // 16):
                sl = pl.ds(r * 16, 16)
                rows[j, sl] = rows[j, sl] * aj
        pltpu.sync_copy(rows, acc_sp.at[didx], add=True)
        return c

    lax.fori_loop(0, _ECH, body, 0)
    plsc.subcore_barrier()

    pltpu.sync_copy(acc_sp.at[pl.ds(nbase, _NPT), :],
                    pout_hbm.at[cid, pl.ds(nbase, _NPT), :])


def _agg(src, dst, alpha, xw):
    f = functools.partial(
        pl.kernel,
        mesh=plsc.VectorSubcoreMesh(core_axis_name="c", subcore_axis_name="s"),
        compiler_params=pltpu.CompilerParams(needs_layout_passes=False, use_tc_tiling_on_sc=False),
        out_type=jax.ShapeDtypeStruct((_NC, _N, _D), jnp.float32),
        scratch_types=[
            pltpu.VMEM((_EPT,), jnp.int32),
            pltpu.VMEM((_EPT,), jnp.int32),
            pltpu.VMEM((_EPT,), jnp.float32),
            pltpu.VMEM((16, _D), jnp.float32),
            pltpu.VMEM_SHARED((_N, _D), jnp.float32),
            pltpu.SemaphoreType.DMA,
        ],
    )(_agg_body)
    return f(src, dst, alpha, xw)


def _fin_body(p_ref, b_ref, o_ref):
    o_ref[...] = jnp.maximum(p_ref[0] + p_ref[1] + b_ref[...][None, :], 0.0)


def _finalize(parts, bias):
    return pl.pallas_call(
        _fin_body,
        out_shape=jax.ShapeDtypeStruct((_N, _D), jnp.float32),
    )(parts, bias)


def kernel(x, edge_index, W, att_src, att_dst, bias):
    src = edge_index[0].astype(jnp.int32)
    dst = edge_index[1].astype(jnp.int32)
    xw, a_src, a_dst, m16 = _dense(x, W, att_src, att_dst)
    w, dparts = _edge_w(src, dst, a_src, a_dst, m16)
    alpha = _alpha(dst, w, dparts)
    parts = _agg(src, dst, alpha, xw)
    return _finalize(parts, bias)


# speedup vs baseline: 29.2712x; 2.0695x over previous
import functools

import jax
import jax.numpy as jnp
from jax import lax
from jax.experimental import pallas as pl
from jax.experimental.pallas import tpu as pltpu
from jax.experimental.pallas import tpu_sc as plsc

_N = 10000
_E = 320000
_D = 128

_NC = 2
_NS = 16
_NW = _NC * _NS
_EPT = _E // _NW
_ECH = _EPT // 16
_NPT = _N // _NS
_NCH = _N // 16

_SC_PARAMS = pltpu.CompilerParams(needs_layout_passes=False,
                                  use_tc_tiling_on_sc=False)
_SC_MESH = plsc.VectorSubcoreMesh(core_axis_name="c", subcore_axis_name="s")


def _bcast_lane(v, j):
    idx = jnp.full((16, 1), j, jnp.int32)
    dn = lax.GatherDimensionNumbers(offset_dims=(), collapsed_slice_dims=(0,),
                                    start_index_map=(0,))
    return lax.gather(v, idx, dn, slice_sizes=(1,),
                      mode=lax.GatherScatterMode.PROMISE_IN_BOUNDS)


def _dense_body(x_ref, w_ref, asv_ref, adv_ref, xw_ref, a_src_ref,
                a_dst_ref, m_ref):
    xw = jnp.dot(x_ref[...], w_ref[...], preferred_element_type=jnp.float32)
    xw_ref[...] = xw
    a_src = jnp.sum(xw * asv_ref[...][None, :], axis=1)
    a_dst = jnp.sum(xw * adv_ref[...][None, :], axis=1)
    a_src_ref[...] = a_src
    a_dst_ref[...] = a_dst
    m = jnp.max(a_src) + jnp.max(a_dst)
    m_ref[...] = jnp.full((16,), m, jnp.float32)


def _dense(x, w, att_src, att_dst):
    return pl.pallas_call(
        _dense_body,
        out_shape=(
            jax.ShapeDtypeStruct((_N, _D), jnp.float32),
            jax.ShapeDtypeStruct((_N,), jnp.float32),
            jax.ShapeDtypeStruct((_N,), jnp.float32),
            jax.ShapeDtypeStruct((16,), jnp.float32),
        ),
    )(x, w, att_src, att_dst)


def _edge_w_body(src_hbm, dst_hbm, asrc_hbm, adst_hbm, m_hbm,
                 w_hbm, dpart_hbm,
                 srcv, dstv, asv, adv, wv, denv, mv):
    cid = lax.axis_index("c")
    sid = lax.axis_index("s")
    gid = sid * _NC + cid
    base = gid * _EPT
    pltpu.sync_copy(src_hbm.at[pl.ds(base, _EPT)], srcv)
    pltpu.sync_copy(dst_hbm.at[pl.ds(base, _EPT)], dstv)
    pltpu.sync_copy(asrc_hbm, asv)
    pltpu.sync_copy(adst_hbm, adv)
    pltpu.sync_copy(m_hbm, mv)
    mvec = mv[...]

    zero = jnp.zeros((16,), jnp.float32)

    def zbody(i, c):
        denv[pl.ds(i * 16, 16)] = zero
        return c

    lax.fori_loop(0, _NCH, zbody, 0)

    def body(i, c):
        sidx = srcv[pl.ds(i * 16, 16)]
        didx = dstv[pl.ds(i * 16, 16)]
        s = plsc.load_gather(asv, [sidx])
        d = plsc.load_gather(adv, [didx])
        e = s + d
        e = jnp.maximum(e, e * 0.2)
        w = jnp.exp(e - mvec)
        wv[pl.ds(i * 16, 16)] = w
        plsc.addupdate_scatter(denv, [didx], w)
        return c

    lax.fori_loop(0, _ECH, body, 0)

    pltpu.sync_copy(wv, w_hbm.at[pl.ds(base, _EPT)])
    pltpu.sync_copy(denv, dpart_hbm.at[gid])


def _edge_w(src, dst, a_src, a_dst, m16):
    f = functools.partial(
        pl.kernel,
        mesh=_SC_MESH,
        compiler_params=_SC_PARAMS,
        out_type=(
            jax.ShapeDtypeStruct((_E,), jnp.float32),
            jax.ShapeDtypeStruct((_NW, _N), jnp.float32),
        ),
        scratch_types=[
            pltpu.VMEM((_EPT,), jnp.int32),
            pltpu.VMEM((_EPT,), jnp.int32),
            pltpu.VMEM((_N,), jnp.float32),
            pltpu.VMEM((_N,), jnp.float32),
            pltpu.VMEM((_EPT,), jnp.float32),
            pltpu.VMEM((_N,), jnp.float32),
            pltpu.VMEM((16,), jnp.float32),
        ],
    )(_edge_w_body)
    return f(src, dst, a_src, a_dst, m16)


def _agg_body(src_hbm, dst_hbm, w_hbm, xw_hbm,
              pout_hbm,
              srcv, dstv, wv, rows0, rows1, acc_sp, sem0, sem1):
    cid = lax.axis_index("c")
    sid = lax.axis_index("s")
    gid = sid * _NC + cid
    base = gid * _EPT
    pltpu.sync_copy(src_hbm.at[pl.ds(base, _EPT)], srcv)
    pltpu.sync_copy(dst_hbm.at[pl.ds(base, _EPT)], dstv)
    pltpu.sync_copy(w_hbm.at[pl.ds(base, _EPT)], wv)

    zerov = jnp.zeros((16,), jnp.float32)
    for a in range(16):
        for b in range(_D // 16):
            rows0[a, pl.ds(b * 16, 16)] = zerov
    nbase = sid * _NPT

    def zc(k, c):
        pltpu.sync_copy(rows0, acc_sp.at[pl.ds(nbase + k * 16, 16), :])
        return c

    lax.fori_loop(0, _NPT // 16, zc, 0)
    pltpu.sync_copy(rows0.at[pl.ds(0, _NPT % 16), :],
                    acc_sp.at[pl.ds(nbase + (_NPT // 16) * 16, _NPT % 16), :])
    plsc.subcore_barrier()

    def fire(i, buf, sem):
        pltpu.async_copy(xw_hbm.at[srcv[pl.ds(i * 16, 16)]], buf, sem)

    def drain(buf, sem):
        pltpu.make_async_copy(xw_hbm.at[pl.ds(0, 16), :], buf, sem).wait()

    def proc(i, buf):
        w_v = wv[pl.ds(i * 16, 16)]
        didx = dstv[pl.ds(i * 16, 16)]
        for j in range(16):
            aj = _bcast_lane(w_v, j)
            for r in range(_D // 16):
                sl = pl.ds(r * 16, 16)
                buf[j, sl] = buf[j, sl] * aj
        pltpu.sync_copy(buf, acc_sp.at[didx], add=True)

    fire(0, rows0, sem0)

    def pair(k, c):
        i0 = 2 * k
        fire(i0 + 1, rows1, sem1)
        drain(rows0, sem0)
        proc(i0, rows0)
        fire(i0 + 2, rows0, sem0)
        drain(rows1, sem1)
        proc(i0 + 1, rows1)
        return c

    lax.fori_loop(0, (_ECH - 1) // 2, pair, 0)
    drain(rows0, sem0)
    proc(_ECH - 1, rows0)
    plsc.subcore_barrier()

    pltpu.sync_copy(acc_sp.at[pl.ds(nbase, _NPT), :],
                    pout_hbm.at[cid, pl.ds(nbase, _NPT), :])


def _agg(src, dst, w, xw):
    f = functools.partial(
        pl.kernel,
        mesh=_SC_MESH,
        compiler_params=_SC_PARAMS,
        out_type=jax.ShapeDtypeStruct((_NC, _N, _D), jnp.float32),
        scratch_types=[
            pltpu.VMEM((_EPT,), jnp.int32),
            pltpu.VMEM((_EPT,), jnp.int32),
            pltpu.VMEM((_EPT,), jnp.float32),
            pltpu.VMEM((16, _D), jnp.float32),
            pltpu.VMEM((16, _D), jnp.float32),
            pltpu.VMEM_SHARED((_N, _D), jnp.float32),
            pltpu.SemaphoreType.DMA,
            pltpu.SemaphoreType.DMA,
        ],
    )(_agg_body)
    return f(src, dst, w, xw)


def _fin_body(p_ref, dpart_ref, b_ref, o_ref):
    denom = jnp.sum(dpart_ref[...], axis=0) + 1e-16
    s = (p_ref[0] + p_ref[1]) / denom[:, None]
    o_ref[...] = jnp.maximum(s + b_ref[...][None, :], 0.0)


def _finalize(parts, dparts, bias):
    return pl.pallas_call(
        _fin_body,
        out_shape=jax.ShapeDtypeStruct((_N, _D), jnp.float32),
    )(parts, dparts, bias)


def kernel(x, edge_index, W, att_src, att_dst, bias):
    src = edge_index[0].astype(jnp.int32)
    dst = edge_index[1].astype(jnp.int32)
    xw, a_src, a_dst, m16 = _dense(x, W, att_src, att_dst)
    w, dparts = _edge_w(src, dst, a_src, a_dst, m16)
    parts = _agg(src, dst, w, xw)
    return _finalize(parts, dparts, bias)


# speedup vs baseline: 49.8611x; 1.7034x over previous
import functools

import jax
import jax.numpy as jnp
from jax import lax
from jax.experimental import pallas as pl
from jax.experimental.pallas import tpu as pltpu
from jax.experimental.pallas import tpu_sc as plsc

_N = 10000
_E = 320000
_D = 128

_NC = 2
_NS = 16
_NW = _NC * _NS
_EPT = _E // _NW
_ECH = _EPT // 16
_NPT = _N // _NS
_NCH = _N // 16

_SC_PARAMS = pltpu.CompilerParams(needs_layout_passes=False,
                                  use_tc_tiling_on_sc=False)
_SC_MESH = plsc.VectorSubcoreMesh(core_axis_name="c", subcore_axis_name="s")


def _bcast_lane(v, j):
    idx = jnp.full((16, 1), j, jnp.int32)
    dn = lax.GatherDimensionNumbers(offset_dims=(), collapsed_slice_dims=(0,),
                                    start_index_map=(0,))
    return lax.gather(v, idx, dn, slice_sizes=(1,),
                      mode=lax.GatherScatterMode.PROMISE_IN_BOUNDS)


def _dense_body(x_ref, w_ref, asv_ref, adv_ref, xw_ref, a_src_ref,
                a_dst_ref, m_ref):
    xw = jnp.dot(x_ref[...], w_ref[...], preferred_element_type=jnp.float32)
    xw_ref[...] = xw
    a_src = jnp.sum(xw * asv_ref[...][None, :], axis=1)
    a_dst = jnp.sum(xw * adv_ref[...][None, :], axis=1)
    a_src_ref[...] = a_src
    a_dst_ref[...] = a_dst
    m = jnp.max(a_src) + jnp.max(a_dst)
    m_ref[...] = jnp.full((16,), m, jnp.float32)


def _dense(x, w, att_src, att_dst):
    return pl.pallas_call(
        _dense_body,
        out_shape=(
            jax.ShapeDtypeStruct((_N, _D), jnp.float32),
            jax.ShapeDtypeStruct((_N,), jnp.float32),
            jax.ShapeDtypeStruct((_N,), jnp.float32),
            jax.ShapeDtypeStruct((16,), jnp.float32),
        ),
    )(x, w, att_src, att_dst)


def _edge_w_body(src_hbm, dst_hbm, asrc_hbm, adst_hbm, m_hbm,
                 w_hbm, dpart_hbm,
                 srcv, dstv, asv, adv, wv, denv, mv):
    cid = lax.axis_index("c")
    sid = lax.axis_index("s")
    gid = sid * _NC + cid
    base = gid * _EPT
    pltpu.sync_copy(src_hbm.at[pl.ds(base, _EPT)], srcv)
    pltpu.sync_copy(dst_hbm.at[pl.ds(base, _EPT)], dstv)
    pltpu.sync_copy(asrc_hbm, asv)
    pltpu.sync_copy(adst_hbm, adv)
    pltpu.sync_copy(m_hbm, mv)
    mvec = mv[...]

    zero = jnp.zeros((16,), jnp.float32)

    def zbody(i, c):
        denv[pl.ds(i * 16, 16)] = zero
        return c

    lax.fori_loop(0, _NCH, zbody, 0)

    def body(i, c):
        sidx = srcv[pl.ds(i * 16, 16)]
        didx = dstv[pl.ds(i * 16, 16)]
        s = plsc.load_gather(asv, [sidx])
        d = plsc.load_gather(adv, [didx])
        e = s + d
        e = jnp.maximum(e, e * 0.2)
        w = jnp.exp(e - mvec)
        wv[pl.ds(i * 16, 16)] = w
        plsc.addupdate_scatter(denv, [didx], w)
        return c

    lax.fori_loop(0, _ECH, body, 0)

    pltpu.sync_copy(wv, w_hbm.at[pl.ds(base, _EPT)])
    pltpu.sync_copy(denv, dpart_hbm.at[gid])


def _edge_w(src, dst, a_src, a_dst, m16):
    f = functools.partial(
        pl.kernel,
        mesh=_SC_MESH,
        compiler_params=_SC_PARAMS,
        out_type=(
            jax.ShapeDtypeStruct((_E,), jnp.float32),
            jax.ShapeDtypeStruct((_NW, _N), jnp.float32),
        ),
        scratch_types=[
            pltpu.VMEM((_EPT,), jnp.int32),
            pltpu.VMEM((_EPT,), jnp.int32),
            pltpu.VMEM((_N,), jnp.float32),
            pltpu.VMEM((_N,), jnp.float32),
            pltpu.VMEM((_EPT,), jnp.float32),
            pltpu.VMEM((_N,), jnp.float32),
            pltpu.VMEM((16,), jnp.float32),
        ],
    )(_edge_w_body)
    return f(src, dst, a_src, a_dst, m16)


_CHW = 80
_NCHK = _EPT // _CHW


def _agg_body(src_hbm, dst_hbm, w_hbm, xw_hbm,
              pout_hbm,
              srcv, dstv, wv, rows0, rows1, acc_sp, sem0, sem1):
    cid = lax.axis_index("c")
    sid = lax.axis_index("s")
    gid = sid * _NC + cid
    pltpu.sync_copy(src_hbm.at[gid], srcv)
    pltpu.sync_copy(dst_hbm.at[gid], dstv)
    pltpu.sync_copy(w_hbm.at[gid], wv)

    zerov = jnp.zeros((16,), jnp.float32)
    for a in range(_CHW):
        for b in range(_D // 16):
            rows0[a, pl.ds(b * 16, 16)] = zerov
    nbase = sid * _NPT

    def zc(k, c):
        pltpu.sync_copy(rows0, acc_sp.at[pl.ds(nbase + k * _CHW, _CHW), :])
        return c

    lax.fori_loop(0, _NPT // _CHW, zc, 0)
    pltpu.sync_copy(rows0.at[pl.ds(0, _NPT % _CHW), :],
                    acc_sp.at[pl.ds(nbase + (_NPT // _CHW) * _CHW,
                                    _NPT % _CHW), :])
    plsc.subcore_barrier()

    def fire(i, buf, sem):
        pltpu.async_copy(xw_hbm.at[srcv.at[i]], buf, sem)

    def drain(buf, sem):
        pltpu.make_async_copy(xw_hbm.at[pl.ds(0, _CHW), :], buf, sem).wait()

    def proc(i, buf):
        def sub(t, c):
            w_v = wv[i, pl.ds(t * 16, 16)]
            for j in range(16):
                aj = _bcast_lane(w_v, j)
                for r in range(_D // 16):
                    sl = pl.ds(r * 16, 16)
                    buf[t * 16 + j, sl] = buf[t * 16 + j, sl] * aj
            return c

        lax.fori_loop(0, _CHW // 16, sub, 0)
        pltpu.sync_copy(buf, acc_sp.at[dstv.at[i]], add=True)

    fire(0, rows0, sem0)

    def pair(k, c):
        i0 = 2 * k
        fire(i0 + 1, rows1, sem1)
        drain(rows0, sem0)
        proc(i0, rows0)
        fire(i0 + 2, rows0, sem0)
        drain(rows1, sem1)
        proc(i0 + 1, rows1)
        return c

    lax.fori_loop(0, (_NCHK - 1) // 2, pair, 0)
    drain(rows0, sem0)
    proc(_NCHK - 1, rows0)
    plsc.subcore_barrier()

    pltpu.sync_copy(acc_sp.at[pl.ds(nbase, _NPT), :],
                    pout_hbm.at[cid, pl.ds(nbase, _NPT), :])


def _agg(src, dst, w, xw):
    f = functools.partial(
        pl.kernel,
        mesh=_SC_MESH,
        compiler_params=_SC_PARAMS,
        out_type=jax.ShapeDtypeStruct((_NC, _N, _D), jnp.float32),
        scratch_types=[
            pltpu.VMEM((_NCHK, _CHW), jnp.int32),
            pltpu.VMEM((_NCHK, _CHW), jnp.int32),
            pltpu.VMEM((_NCHK, _CHW), jnp.float32),
            pltpu.VMEM((_CHW, _D), jnp.float32),
            pltpu.VMEM((_CHW, _D), jnp.float32),
            pltpu.VMEM_SHARED((_N, _D), jnp.float32),
            pltpu.SemaphoreType.DMA,
            pltpu.SemaphoreType.DMA,
        ],
    )(_agg_body)
    src3 = src.reshape(_NW, _NCHK, _CHW)
    dst3 = dst.reshape(_NW, _NCHK, _CHW)
    w3 = w.reshape(_NW, _NCHK, _CHW)
    return f(src3, dst3, w3, xw)


def _fin_body(p_ref, dpart_ref, b_ref, o_ref):
    denom = jnp.sum(dpart_ref[...], axis=0) + 1e-16
    s = (p_ref[0] + p_ref[1]) / denom[:, None]
    o_ref[...] = jnp.maximum(s + b_ref[...][None, :], 0.0)


def _finalize(parts, dparts, bias):
    return pl.pallas_call(
        _fin_body,
        out_shape=jax.ShapeDtypeStruct((_N, _D), jnp.float32),
    )(parts, dparts, bias)


def kernel(x, edge_index, W, att_src, att_dst, bias):
    src = edge_index[0].astype(jnp.int32)
    dst = edge_index[1].astype(jnp.int32)
    xw, a_src, a_dst, m16 = _dense(x, W, att_src, att_dst)
    w, dparts = _edge_w(src, dst, a_src, a_dst, m16)
    parts = _agg(src, dst, w, xw)
    return _finalize(parts, dparts, bias)
